# baseline (speedup 1.0000x reference)
import jax
import jax.numpy as jnp
from jax import lax
from jax.experimental import pallas as pl
from jax.experimental.pallas import tpu as pltpu
from jax.experimental.pallas import tpu_sc as plsc

N_NODES = 10000
IN_F = 128
HID = 128
NCLS = 16
N_EDGES = 320000

NC = 2
NS = 16
NW = NC * NS
CHUNK = 128
STRIPE = 640
ROW_PAD = NS * STRIPE

E_AUG = N_EDGES + N_NODES
CPT = 84
E_PAD = CPT * CHUNK * NW
NGRP = 3
G = CPT // NGRP

_MESH = plsc.VectorSubcoreMesh(core_axis_name="c", subcore_axis_name="s")


def _sc_degree(dst3d, ones16, zdeg):
    def body(dst_hbm, ones_hbm, z_hbm, out_hbm, dst_v, ones_v, acc):
        c = lax.axis_index("c")
        s = lax.axis_index("s")
        wid = c * NS + s
        pltpu.sync_copy(ones_hbm, ones_v)
        pltpu.sync_copy(z_hbm, acc.at[pl.ds(s * STRIPE, STRIPE)])
        plsc.subcore_barrier()

        for grp in range(NGRP):
            pltpu.sync_copy(dst_hbm.at[wid * NGRP + grp], dst_v)

            def step(j, carry):
                pltpu.sync_copy(ones_v, acc.at[dst_v.at[j]], add=True)
                return carry

            lax.fori_loop(0, G, step, 0)
        plsc.subcore_barrier()
        pltpu.sync_copy(acc.at[pl.ds(s * STRIPE, STRIPE)],
                        out_hbm.at[c, pl.ds(s * STRIPE, STRIPE)])

    k = pl.kernel(
        body,
        out_type=jax.ShapeDtypeStruct((NC, ROW_PAD, 16), jnp.float32),
        mesh=_MESH,
        scratch_types=[
            pltpu.VMEM((G, CHUNK), jnp.int32),
            pltpu.VMEM((CHUNK, 16), jnp.float32),
            pltpu.VMEM_SHARED((ROW_PAD, 16), jnp.float32),
        ],
    )
    return k(dst3d, ones16, zdeg)


def _sc_edge_pass(h, src3d, dst3d, zrows):
    def body(h_hbm, src_hbm, dst_hbm, z_hbm, out_hbm, src_g, dst_g,
             rows0, rows1, sem0, sem1, ssem0, ssem1, acc):
        c = lax.axis_index("c")
        s = lax.axis_index("s")
        wid = c * NS + s
        rows = (rows0, rows1)
        sems = (sem0, sem1)
        ssems = (ssem0, ssem1)
        pltpu.sync_copy(z_hbm, acc.at[pl.ds(s * STRIPE, STRIPE)])
        plsc.subcore_barrier()

        for grp in range(NGRP):
            pltpu.sync_copy(src_hbm.at[wid * NGRP + grp], src_g)
            pltpu.sync_copy(dst_hbm.at[wid * NGRP + grp], dst_g)
            for b in range(2):
                pltpu.async_copy(h_hbm.at[src_g.at[b]], rows[b], sems[b])

            def step(t, carry):
                for b in range(2):
                    j = t * 2 + b
                    pltpu.make_async_copy(h_hbm.at[src_g.at[j]], rows[b],
                                          sems[b]).wait()
                    pltpu.async_copy(rows[b], acc.at[dst_g.at[j]], ssems[b],
                                     add=True)
                for b in range(2):
                    j = t * 2 + b
                    pltpu.make_async_copy(rows[b], acc.at[dst_g.at[j]],
                                          ssems[b]).wait()
                    pltpu.async_copy(h_hbm.at[src_g.at[j + 2]], rows[b],
                                     sems[b])
                return carry

            lax.fori_loop(0, G // 2 - 1, step, 0)
            for b in range(2):
                j = G - 2 + b
                pltpu.make_async_copy(h_hbm.at[src_g.at[j]], rows[b],
                                      sems[b]).wait()
                pltpu.sync_copy(rows[b], acc.at[dst_g.at[j]], add=True)
        plsc.subcore_barrier()
        pltpu.sync_copy(acc.at[pl.ds(s * STRIPE, STRIPE)],
                        out_hbm.at[c, pl.ds(s * STRIPE, STRIPE)])

    k = pl.kernel(
        body,
        out_type=jax.ShapeDtypeStruct((NC, ROW_PAD, HID), jnp.float32),
        mesh=_MESH,
        scratch_types=[
            pltpu.VMEM((G, CHUNK), jnp.int32),
            pltpu.VMEM((G, CHUNK), jnp.int32),
            pltpu.VMEM((CHUNK, HID), jnp.float32),
            pltpu.VMEM((CHUNK, HID), jnp.float32),
            pltpu.SemaphoreType.DMA,
            pltpu.SemaphoreType.DMA,
            pltpu.SemaphoreType.DMA,
            pltpu.SemaphoreType.DMA,
            pltpu.VMEM_SHARED((ROW_PAD, HID), jnp.float32),
        ],
    )
    return k(h, src3d, dst3d, zrows)


BLK = 1000


def _tc_mm(x, w1t):
    def body(x_ref, w_ref, o_ref):
        o_ref[...] = jnp.dot(x_ref[...], w_ref[...],
                             preferred_element_type=jnp.float32)

    return pl.pallas_call(
        body,
        grid=(N_NODES // BLK,),
        in_specs=[
            pl.BlockSpec((BLK, IN_F), lambda i: (i, 0)),
            pl.BlockSpec((IN_F, HID), lambda i: (0, 0)),
        ],
        out_specs=pl.BlockSpec((BLK, HID), lambda i: (i, 0)),
        out_shape=jax.ShapeDtypeStruct((N_NODES, HID), jnp.float32),
    )(x, w1t)


def _tc_stage1(mm, degp):
    def body(mm_ref, d0_ref, d1_ref, h_ref, dis_ref):
        deg = d0_ref[0, :, 0] + d1_ref[0, :, 0]
        dis = lax.rsqrt(deg)
        h_ref[...] = mm_ref[...] * dis[:, None]
        dis_ref[...] = jnp.broadcast_to(dis[:, None], (BLK, 16))

    return pl.pallas_call(
        body,
        grid=(N_NODES // BLK,),
        in_specs=[
            pl.BlockSpec((BLK, HID), lambda i: (i, 0)),
            pl.BlockSpec((1, BLK, 16), lambda i: (0, i, 0)),
            pl.BlockSpec((1, BLK, 16), lambda i: (1, i, 0)),
        ],
        out_specs=[
            pl.BlockSpec((BLK, HID), lambda i: (i, 0)),
            pl.BlockSpec((BLK, 16), lambda i: (i, 0)),
        ],
        out_shape=[
            jax.ShapeDtypeStruct((N_NODES, HID), jnp.float32),
            jax.ShapeDtypeStruct((N_NODES, 16), jnp.float32),
        ],
    )(mm, degp, degp)


def _tc_stage2(q, dis16, b, wt):
    def body(q0_ref, q1_ref, dis_ref, b_ref, w_ref, out_ref):
        dis = dis_ref[:, 0]
        o = (q0_ref[0] + q1_ref[0]) * dis[:, None] + b_ref[...]
        h = jnp.maximum(o, 0.0)
        hw = jnp.dot(h, w_ref[...], preferred_element_type=jnp.float32)
        out_ref[...] = hw * dis[:, None]

    return pl.pallas_call(
        body,
        grid=(N_NODES // BLK,),
        in_specs=[
            pl.BlockSpec((1, BLK, HID), lambda i: (0, i, 0)),
            pl.BlockSpec((1, BLK, HID), lambda i: (1, i, 0)),
            pl.BlockSpec((BLK, 16), lambda i: (i, 0)),
            pl.BlockSpec((1, HID), lambda i: (0, 0)),
            pl.BlockSpec((HID, HID), lambda i: (0, 0)),
        ],
        out_specs=pl.BlockSpec((BLK, HID), lambda i: (i, 0)),
        out_shape=jax.ShapeDtypeStruct((N_NODES, HID), jnp.float32),
    )(q, q, dis16, b, wt)


def _tc_stage3(r, dis16, b, wlt, bl):
    def body(r0_ref, r1_ref, dis_ref, b_ref, w_ref, bl_ref, out_ref):
        dis = dis_ref[:, 0]
        o = (r0_ref[0] + r1_ref[0]) * dis[:, None] + b_ref[...]
        h = jnp.maximum(o, 0.0)
        logits = jnp.dot(h, w_ref[...],
                         preferred_element_type=jnp.float32) + bl_ref[...]
        m = jnp.max(logits, axis=1, keepdims=True)
        e = jnp.exp(logits - m)
        lse = jnp.log(jnp.sum(e, axis=1, keepdims=True)) + m
        out_ref[...] = logits - lse

    return pl.pallas_call(
        body,
        grid=(N_NODES // BLK,),
        in_specs=[
            pl.BlockSpec((1, BLK, HID), lambda i: (0, i, 0)),
            pl.BlockSpec((1, BLK, HID), lambda i: (1, i, 0)),
            pl.BlockSpec((BLK, 16), lambda i: (i, 0)),
            pl.BlockSpec((1, HID), lambda i: (0, 0)),
            pl.BlockSpec((HID, NCLS), lambda i: (0, 0)),
            pl.BlockSpec((1, NCLS), lambda i: (0, 0)),
        ],
        out_specs=pl.BlockSpec((BLK, NCLS), lambda i: (i, 0)),
        out_shape=jax.ShapeDtypeStruct((N_NODES, NCLS), jnp.float32),
    )(r, r, dis16, b, wlt, bl)


def kernel(x, edge_index, W1, b1, W2, b2, Wl, bl):
    ei = edge_index.astype(jnp.int32)
    loop = jnp.arange(N_NODES, dtype=jnp.int32)
    npad = E_PAD - E_AUG
    padlane = jnp.arange(npad, dtype=jnp.int32) % 128
    src = jnp.concatenate([ei[0], loop, padlane])
    dst = jnp.concatenate([ei[1], loop, N_NODES + padlane])
    src3d = src.reshape(NW * NGRP, G, CHUNK)
    dst3d = dst.reshape(NW * NGRP, G, CHUNK)

    ones16 = jnp.ones((CHUNK, 16), jnp.float32)
    zdeg = jnp.zeros((STRIPE, 16), jnp.float32)
    zrows = jnp.zeros((STRIPE, HID), jnp.float32)

    mm = _tc_mm(x, W1.T)
    degp = _sc_degree(dst3d, ones16, zdeg)
    h1, dis16 = _tc_stage1(mm, degp)
    q = _sc_edge_pass(h1, src3d, dst3d, zrows)
    h2 = _tc_stage2(q, dis16, b1.reshape(1, HID), W2.T)
    r = _sc_edge_pass(h2, src3d, dst3d, zrows)
    return _tc_stage3(r, dis16, b2.reshape(1, HID), Wl.T,
                      bl.reshape(1, NCLS))


# speedup vs baseline: 1.2088x; 1.2088x over previous
import jax
import jax.numpy as jnp
from jax import lax
from jax.experimental import pallas as pl
from jax.experimental.pallas import tpu as pltpu
from jax.experimental.pallas import tpu_sc as plsc

N_NODES = 10000
IN_F = 128
HID = 128
NCLS = 16
N_EDGES = 320000

NC = 2
NS = 16
NW = NC * NS
CHUNK = 128
STRIPE = 640
ROW_PAD = NS * STRIPE

E_AUG = N_EDGES + N_NODES
CPT = 84
E_PAD = CPT * CHUNK * NW
NGRP = 3
G = CPT // NGRP

_MESH = plsc.VectorSubcoreMesh(core_axis_name="c", subcore_axis_name="s")


def _sc_degree(dst3d, ones16, zdeg):
    def body(dst_hbm, ones_hbm, z_hbm, out_hbm, dst_v, ones_v, acc):
        c = lax.axis_index("c")
        s = lax.axis_index("s")
        wid = c * NS + s
        pltpu.sync_copy(ones_hbm, ones_v)
        pltpu.sync_copy(z_hbm, acc.at[pl.ds(s * STRIPE, STRIPE)])
        plsc.subcore_barrier()

        for grp in range(NGRP):
            pltpu.sync_copy(dst_hbm.at[wid * NGRP + grp], dst_v)

            def step(j, carry):
                pltpu.sync_copy(ones_v, acc.at[dst_v.at[j]], add=True)
                return carry

            lax.fori_loop(0, G, step, 0)
        plsc.subcore_barrier()
        pltpu.sync_copy(acc.at[pl.ds(s * STRIPE, STRIPE)],
                        out_hbm.at[c, pl.ds(s * STRIPE, STRIPE)])

    k = pl.kernel(
        body,
        out_type=jax.ShapeDtypeStruct((NC, ROW_PAD, 16), jnp.float32),
        mesh=_MESH,
        scratch_types=[
            pltpu.VMEM((G, CHUNK), jnp.int32),
            pltpu.VMEM((CHUNK, 16), jnp.float32),
            pltpu.VMEM_SHARED((ROW_PAD, 16), jnp.float32),
        ],
    )
    return k(dst3d, ones16, zdeg)


def _sc_edge_pass(h, src3d, dst3d, zrows):
    def body(h_hbm, src_hbm, dst_hbm, z_hbm, out_hbm, src_g, dst_g,
             rows0, rows1, sem0, sem1, acc):
        c = lax.axis_index("c")
        s = lax.axis_index("s")
        wid = c * NS + s
        rows = (rows0, rows1)
        sems = (sem0, sem1)
        pltpu.sync_copy(z_hbm, acc.at[pl.ds(s * STRIPE, STRIPE)])
        plsc.subcore_barrier()

        for grp in range(NGRP):
            pltpu.sync_copy(src_hbm.at[wid * NGRP + grp], src_g)
            pltpu.sync_copy(dst_hbm.at[wid * NGRP + grp], dst_g)
            for b in range(2):
                pltpu.async_copy(h_hbm.at[src_g.at[b]], rows[b], sems[b])

            def step(t, carry):
                for b in range(2):
                    j = t * 2 + b
                    pltpu.make_async_copy(h_hbm.at[src_g.at[j]], rows[b],
                                          sems[b]).wait()
                    pltpu.sync_copy(rows[b], acc.at[dst_g.at[j]], add=True)
                    pltpu.async_copy(h_hbm.at[src_g.at[j + 2]], rows[b],
                                     sems[b])
                return carry

            lax.fori_loop(0, G // 2 - 1, step, 0)
            for b in range(2):
                j = G - 2 + b
                pltpu.make_async_copy(h_hbm.at[src_g.at[j]], rows[b],
                                      sems[b]).wait()
                pltpu.sync_copy(rows[b], acc.at[dst_g.at[j]], add=True)
        plsc.subcore_barrier()
        pltpu.sync_copy(acc.at[pl.ds(s * STRIPE, STRIPE)],
                        out_hbm.at[c, pl.ds(s * STRIPE, STRIPE)])

    k = pl.kernel(
        body,
        out_type=jax.ShapeDtypeStruct((NC, ROW_PAD, HID), jnp.float32),
        mesh=_MESH,
        scratch_types=[
            pltpu.VMEM((G, CHUNK), jnp.int32),
            pltpu.VMEM((G, CHUNK), jnp.int32),
            pltpu.VMEM((CHUNK, HID), jnp.float32),
            pltpu.VMEM((CHUNK, HID), jnp.float32),
            pltpu.SemaphoreType.DMA,
            pltpu.SemaphoreType.DMA,
            pltpu.VMEM_SHARED((ROW_PAD, HID), jnp.float32),
        ],
    )
    return k(h, src3d, dst3d, zrows)


BLK = 1000


def _tc_mm(x, w1t):
    def body(x_ref, w_ref, o_ref):
        o_ref[...] = jnp.dot(x_ref[...], w_ref[...],
                             preferred_element_type=jnp.float32)

    return pl.pallas_call(
        body,
        grid=(N_NODES // BLK,),
        in_specs=[
            pl.BlockSpec((BLK, IN_F), lambda i: (i, 0)),
            pl.BlockSpec((IN_F, HID), lambda i: (0, 0)),
        ],
        out_specs=pl.BlockSpec((BLK, HID), lambda i: (i, 0)),
        out_shape=jax.ShapeDtypeStruct((N_NODES, HID), jnp.float32),
    )(x, w1t)


def _tc_stage1(mm, degp):
    def body(mm_ref, d0_ref, d1_ref, h_ref, dis_ref):
        deg = d0_ref[0, :, 0] + d1_ref[0, :, 0]
        dis = lax.rsqrt(deg)
        h_ref[...] = mm_ref[...] * dis[:, None]
        dis_ref[...] = jnp.broadcast_to(dis[:, None], (BLK, 16))

    return pl.pallas_call(
        body,
        grid=(N_NODES // BLK,),
        in_specs=[
            pl.BlockSpec((BLK, HID), lambda i: (i, 0)),
            pl.BlockSpec((1, BLK, 16), lambda i: (0, i, 0)),
            pl.BlockSpec((1, BLK, 16), lambda i: (1, i, 0)),
        ],
        out_specs=[
            pl.BlockSpec((BLK, HID), lambda i: (i, 0)),
            pl.BlockSpec((BLK, 16), lambda i: (i, 0)),
        ],
        out_shape=[
            jax.ShapeDtypeStruct((N_NODES, HID), jnp.float32),
            jax.ShapeDtypeStruct((N_NODES, 16), jnp.float32),
        ],
    )(mm, degp, degp)


def _tc_stage2(q, dis16, b, wt):
    def body(q0_ref, q1_ref, dis_ref, b_ref, w_ref, out_ref):
        dis = dis_ref[:, 0]
        o = (q0_ref[0] + q1_ref[0]) * dis[:, None] + b_ref[...]
        h = jnp.maximum(o, 0.0)
        hw = jnp.dot(h, w_ref[...], preferred_element_type=jnp.float32)
        out_ref[...] = hw * dis[:, None]

    return pl.pallas_call(
        body,
        grid=(N_NODES // BLK,),
        in_specs=[
            pl.BlockSpec((1, BLK, HID), lambda i: (0, i, 0)),
            pl.BlockSpec((1, BLK, HID), lambda i: (1, i, 0)),
            pl.BlockSpec((BLK, 16), lambda i: (i, 0)),
            pl.BlockSpec((1, HID), lambda i: (0, 0)),
            pl.BlockSpec((HID, HID), lambda i: (0, 0)),
        ],
        out_specs=pl.BlockSpec((BLK, HID), lambda i: (i, 0)),
        out_shape=jax.ShapeDtypeStruct((N_NODES, HID), jnp.float32),
    )(q, q, dis16, b, wt)


def _tc_stage3(r, dis16, b, wlt, bl):
    def body(r0_ref, r1_ref, dis_ref, b_ref, w_ref, bl_ref, out_ref):
        dis = dis_ref[:, 0]
        o = (r0_ref[0] + r1_ref[0]) * dis[:, None] + b_ref[...]
        h = jnp.maximum(o, 0.0)
        logits = jnp.dot(h, w_ref[...],
                         preferred_element_type=jnp.float32) + bl_ref[...]
        m = jnp.max(logits, axis=1, keepdims=True)
        e = jnp.exp(logits - m)
        lse = jnp.log(jnp.sum(e, axis=1, keepdims=True)) + m
        out_ref[...] = logits - lse

    return pl.pallas_call(
        body,
        grid=(N_NODES // BLK,),
        in_specs=[
            pl.BlockSpec((1, BLK, HID), lambda i: (0, i, 0)),
            pl.BlockSpec((1, BLK, HID), lambda i: (1, i, 0)),
            pl.BlockSpec((BLK, 16), lambda i: (i, 0)),
            pl.BlockSpec((1, HID), lambda i: (0, 0)),
            pl.BlockSpec((HID, NCLS), lambda i: (0, 0)),
            pl.BlockSpec((1, NCLS), lambda i: (0, 0)),
        ],
        out_specs=pl.BlockSpec((BLK, NCLS), lambda i: (i, 0)),
        out_shape=jax.ShapeDtypeStruct((N_NODES, NCLS), jnp.float32),
    )(r, r, dis16, b, wlt, bl)


def kernel(x, edge_index, W1, b1, W2, b2, Wl, bl):
    ei = edge_index.astype(jnp.int32)
    loop = jnp.arange(N_NODES, dtype=jnp.int32)
    npad = E_PAD - E_AUG
    padlane = jnp.arange(npad, dtype=jnp.int32) % 128
    src = jnp.concatenate([ei[0], loop, padlane])
    dst = jnp.concatenate([ei[1], loop, N_NODES + padlane])
    src3d = src.reshape(NW * NGRP, G, CHUNK)
    dst3d = dst.reshape(NW * NGRP, G, CHUNK)

    ones16 = jnp.ones((CHUNK, 16), jnp.float32)
    zdeg = jnp.zeros((STRIPE, 16), jnp.float32)
    zrows = jnp.zeros((STRIPE, HID), jnp.float32)

    mm = _tc_mm(x, W1.T)
    degp = _sc_degree(dst3d, ones16, zdeg)
    h1, dis16 = _tc_stage1(mm, degp)
    q = _sc_edge_pass(h1, src3d, dst3d, zrows)
    h2 = _tc_stage2(q, dis16, b1.reshape(1, HID), W2.T)
    r = _sc_edge_pass(h2, src3d, dst3d, zrows)
    return _tc_stage3(r, dis16, b2.reshape(1, HID), Wl.T,
                      bl.reshape(1, NCLS))
